# Initial kernel scaffold; baseline (speedup 1.0000x reference)
import functools

import jax
import jax.numpy as jnp
from jax import lax
from jax.experimental import pallas as pl
from jax.experimental.pallas import tpu as pltpu
from jax.experimental.pallas import tpu_sc as plsc

N = 10000
E = 320000
D = 128
NSUB = 16
EPT = E // NSUB
C = 80
NCH = EPT // C
RPT = 624
NZ = RPT // 16


def _rsqrt_newton(d):
    i = lax.bitcast_convert_type(d, jnp.int32)
    y = lax.bitcast_convert_type(jnp.int32(0x5F3759DF) - (i >> 1), jnp.float32)
    for _ in range(3):
        y = y * (1.5 - 0.5 * d * y * y)
    return y


def _make_sc_kernel():
    mesh = plsc.VectorSubcoreMesh(core_axis_name="c", subcore_axis_name="s")

    @functools.partial(
        pl.kernel,
        out_type=[
            jax.ShapeDtypeStruct((2, N, D), jnp.float32),
            jax.ShapeDtypeStruct((2, N), jnp.float32),
        ],
        mesh=mesh,
        scratch_types=[
            pltpu.VMEM((C,), jnp.int32),
            pltpu.VMEM((C,), jnp.int32),
            pltpu.VMEM((C,), jnp.float32),
            pltpu.VMEM((C,), jnp.float32),
            pltpu.VMEM((16,), jnp.float32),
            pltpu.VMEM((C, D), jnp.float32),
            pltpu.VMEM((N,), jnp.float32),
            pltpu.VMEM((16, D), jnp.float32),
            pltpu.VMEM((16,), jnp.float32),
            pltpu.VMEM_SHARED((N, D), jnp.float32),
            pltpu.VMEM_SHARED((N,), jnp.float32),
            pltpu.SemaphoreType.DMA,
        ],
    )
    def sc_kernel(x_hbm, ei_hbm, ef_hbm, h_out, deg_out,
                  row_v, col_v, ef_v, ew_v, norm_v, rows_v, dis_v,
                  z16r, z16d, h_sh, deg_sh, sem):
        c = lax.axis_index("c")
        s = lax.axis_index("s")
        zero16 = jnp.zeros((16,), jnp.float32)

        for i in range(16):
            for k in range(8):
                z16r[i, pl.ds(k * 16, 16)] = zero16
        z16d[...] = zero16
        rbase = s * RPT
        nz = NZ + (s == NSUB - 1).astype(jnp.int32)

        def zero_step(t, carry):
            off = rbase + t * 16
            pltpu.sync_copy(z16r, h_sh.at[pl.ds(off, 16)])
            pltpu.sync_copy(z16d, deg_sh.at[pl.ds(off, 16)])
            return carry

        lax.fori_loop(0, nz, zero_step, 0)
        plsc.subcore_barrier()

        ebase = s * EPT

        def deg_step(t, carry):
            base = ebase + t * C
            pltpu.sync_copy(ei_hbm.at[1, pl.ds(base, C)], col_v)
            pltpu.sync_copy(ef_hbm.at[c, pl.ds(base, C)], ef_v)
            for g in range(C // 16):
                sl = pl.ds(g * 16, 16)
                ew_v[sl] = jnp.abs(ef_v[sl])
            pltpu.sync_copy(ew_v, deg_sh.at[col_v], add=True)
            return carry

        lax.fori_loop(0, NCH, deg_step, 0)
        plsc.subcore_barrier()

        pltpu.sync_copy(deg_sh, dis_v)

        @pl.when(s == 0)
        def _():
            pltpu.sync_copy(deg_sh, deg_out.at[c])

        def dis_step(i, carry):
            sl = pl.ds(i * 16, 16)
            d = dis_v[sl] + 1.0
            dis_v[sl] = _rsqrt_newton(d)
            return carry

        lax.fori_loop(0, N // 16, dis_step, 0)

        def edge_step(t, carry):
            base = ebase + t * C
            pltpu.sync_copy(ei_hbm.at[0, pl.ds(base, C)], row_v)
            pltpu.sync_copy(ei_hbm.at[1, pl.ds(base, C)], col_v)
            pltpu.sync_copy(ef_hbm.at[c, pl.ds(base, C)], ef_v)
            pltpu.async_copy(x_hbm.at[row_v], rows_v, sem).wait()
            for g in range(C // 16):
                sl = pl.ds(g * 16, 16)
                r16 = row_v[sl]
                c16 = col_v[sl]
                e16 = jnp.abs(ef_v[sl])
                dr = plsc.load_gather(dis_v, [r16])
                dc = plsc.load_gather(dis_v, [c16])
                norm_v[...] = dr * e16 * dc
                for l in range(16):
                    spl = plsc.load_gather(norm_v, [jnp.full((16,), l, jnp.int32)])
                    j = g * 16 + l
                    for k in range(8):
                        fs = pl.ds(k * 16, 16)
                        rows_v[j, fs] = rows_v[j, fs] * spl
            pltpu.sync_copy(rows_v, h_sh.at[col_v], add=True)
            return carry

        lax.fori_loop(0, NCH, edge_step, 0)
        plsc.subcore_barrier()

        def wb_step(t, carry):
            off = rbase + t * 16
            pltpu.sync_copy(h_sh.at[pl.ds(off, 16)], h_out.at[c, pl.ds(off, 16)])
            return carry

        lax.fori_loop(0, nz, wb_step, 0)

    return sc_kernel


_sc_kernel = _make_sc_kernel()

BLK = 1000


def _tc_body(h_ref, deg_ref, x_ref, w_ref, b_ref, o_ref):
    rinv = 1.0 / (deg_ref[...] + 1.0)
    x = x_ref[...]
    t1 = h_ref[0] + x * rinv[0][:, None]
    t2 = h_ref[1] + x * rinv[1][:, None]
    o1 = jnp.dot(t1, w_ref[:, :D], preferred_element_type=jnp.float32) + b_ref[0, :D]
    o2 = jnp.dot(t2, w_ref[:, D:], preferred_element_type=jnp.float32) + b_ref[0, D:]
    o_ref[:, 0:D] = o1
    o_ref[:, D:2 * D] = o2
    o_ref[:, 2 * D:3 * D] = o1
    o_ref[:, 3 * D:4 * D] = o2


@jax.jit
def kernel(x, edge_index, edge_feat, W1, b1, W2, b2, W3, b3, W4, b4):
    ef_t = jnp.transpose(edge_feat[:, :2])
    h, deg = _sc_kernel(x, edge_index, ef_t)

    w_cat = jnp.concatenate([W1, W2], axis=1)
    b_cat = jnp.concatenate([b1, b2])[None, :]

    out = pl.pallas_call(
        _tc_body,
        grid=(N // BLK,),
        in_specs=[
            pl.BlockSpec((2, BLK, D), lambda i: (0, i, 0)),
            pl.BlockSpec((2, BLK), lambda i: (0, i)),
            pl.BlockSpec((BLK, D), lambda i: (i, 0)),
            pl.BlockSpec((D, 2 * D), lambda i: (0, 0)),
            pl.BlockSpec((1, 2 * D), lambda i: (0, 0)),
        ],
        out_specs=pl.BlockSpec((BLK, 4 * D), lambda i: (i, 0)),
        out_shape=jax.ShapeDtypeStruct((N, 4 * D), jnp.float32),
    )(h, deg, x, w_cat, b_cat)
    return out


# speedup vs baseline: 15.9023x; 15.9023x over previous
import functools

import jax
import jax.numpy as jnp
from jax import lax
from jax.experimental import pallas as pl
from jax.experimental.pallas import tpu as pltpu
from jax.experimental.pallas import tpu_sc as plsc

N = 10000
E = 320000
D = 128
NSUB = 16
C = 128
NCHT = E // C // NSUB
NP = 10240
RPT = 624
NZ = RPT // 16


def _rsqrt_newton(d):
    i = lax.bitcast_convert_type(d, jnp.int32)
    y = lax.bitcast_convert_type(jnp.int32(0x5F3759DF) - (i >> 1), jnp.float32)
    for _ in range(3):
        y = y * (1.5 - 0.5 * d * y * y)
    return y


def _make_sc_kernel():
    mesh = plsc.VectorSubcoreMesh(core_axis_name="c", subcore_axis_name="s")

    @functools.partial(
        pl.kernel,
        out_type=[
            jax.ShapeDtypeStruct((2, N, D), jnp.float32),
            jax.ShapeDtypeStruct((2, 1, NP), jnp.float32),
        ],
        mesh=mesh,
        scratch_types=[
            pltpu.VMEM((C,), jnp.int32),
            pltpu.VMEM((C,), jnp.int32),
            pltpu.VMEM((C,), jnp.float32),
            pltpu.VMEM((C,), jnp.float32),
            pltpu.VMEM((C, D), jnp.float32),
            pltpu.VMEM((NP,), jnp.float32),
            pltpu.VMEM((16, D), jnp.float32),
            pltpu.VMEM((128,), jnp.float32),
            pltpu.VMEM_SHARED((N, D), jnp.float32),
            pltpu.VMEM_SHARED((NP,), jnp.float32),
            pltpu.SemaphoreType.DMA,
        ],
        compiler_params=pltpu.CompilerParams(needs_layout_passes=False),
    )
    def sc_kernel(x_hbm, row_hbm, col_hbm, ef_hbm, h_out, deg_out,
                  row_v, col_v, ef_v, ew_v, rows_v, dis_v,
                  z16r, z128d, h_sh, deg_sh, sem):
        c = lax.axis_index("c")
        s = lax.axis_index("s")
        zero16 = jnp.zeros((16,), jnp.float32)

        for i in range(16):
            for k in range(8):
                z16r[i, pl.ds(k * 16, 16)] = zero16
        for k in range(8):
            z128d[pl.ds(k * 16, 16)] = zero16
        rbase = s * RPT
        nz = NZ + (s == NSUB - 1).astype(jnp.int32)

        def zero_step(t, carry):
            off = rbase + t * 16
            pltpu.sync_copy(z16r, h_sh.at[pl.ds(off, 16)])
            return carry

        lax.fori_loop(0, nz, zero_step, 0)
        for t in range(5):
            pltpu.sync_copy(z128d, deg_sh.at[pl.ds((s * 5 + t) * 128, 128)])
        plsc.subcore_barrier()

        nch = NCHT + (s < 4).astype(jnp.int32)

        def deg_step(t, carry):
            base = (s + NSUB * t) * C
            pltpu.sync_copy(col_hbm.at[pl.ds(base, C)], col_v)
            pltpu.sync_copy(ef_hbm.at[c, 0, pl.ds(base, C)], ef_v)
            for g in range(C // 16):
                sl = pl.ds(g * 16, 16)
                ew_v[sl] = jnp.abs(ef_v[sl])
            pltpu.sync_copy(ew_v, deg_sh.at[col_v], add=True)
            return carry

        lax.fori_loop(0, nch, deg_step, 0)
        plsc.subcore_barrier()

        pltpu.sync_copy(deg_sh, dis_v)

        @pl.when(s == 0)
        def _():
            pltpu.sync_copy(deg_sh, deg_out.at[c, 0])

        def dis_step(i, carry):
            sl = pl.ds(i * 16, 16)
            d = dis_v[sl] + 1.0
            dis_v[sl] = _rsqrt_newton(d)
            return carry

        lax.fori_loop(0, NP // 16, dis_step, 0)

        def edge_step(t, carry):
            base = (s + NSUB * t) * C
            pltpu.sync_copy(row_hbm.at[pl.ds(base, C)], row_v)
            pltpu.sync_copy(col_hbm.at[pl.ds(base, C)], col_v)
            pltpu.sync_copy(ef_hbm.at[c, 0, pl.ds(base, C)], ef_v)
            pltpu.async_copy(x_hbm.at[row_v], rows_v, sem).wait()
            for g in range(C // 16):
                sl = pl.ds(g * 16, 16)
                r16 = row_v[sl]
                c16 = col_v[sl]
                e16 = jnp.abs(ef_v[sl])
                dr = plsc.load_gather(dis_v, [r16])
                dc = plsc.load_gather(dis_v, [c16])
                norm16 = dr * e16 * dc
                for l in range(16):
                    spl = jnp.full((16,), norm16[l], jnp.float32)
                    j = g * 16 + l
                    for k in range(8):
                        fs = pl.ds(k * 16, 16)
                        rows_v[j, fs] = rows_v[j, fs] * spl
            pltpu.sync_copy(rows_v, h_sh.at[col_v], add=True)
            return carry

        lax.fori_loop(0, nch, edge_step, 0)
        plsc.subcore_barrier()

        def wb_step(t, carry):
            off = rbase + t * 16
            pltpu.sync_copy(h_sh.at[pl.ds(off, 16)], h_out.at[c, pl.ds(off, 16)])
            return carry

        lax.fori_loop(0, nz, wb_step, 0)

    return sc_kernel


_sc_kernel = _make_sc_kernel()

BLK = 1000


def _tc_body(h_ref, deg_ref, x_ref, w_ref, b_ref, o_ref):
    rinv = 1.0 / (deg_ref[...] + 1.0)
    x = x_ref[...]
    t1 = h_ref[0] + x * rinv[:, 0:1]
    t2 = h_ref[1] + x * rinv[:, 1:2]
    o1 = jnp.dot(t1, w_ref[:, :D], preferred_element_type=jnp.float32) + b_ref[0, :D]
    o2 = jnp.dot(t2, w_ref[:, D:], preferred_element_type=jnp.float32) + b_ref[0, D:]
    o_ref[:, 0:D] = o1
    o_ref[:, D:2 * D] = o2
    o_ref[:, 2 * D:3 * D] = o1
    o_ref[:, 3 * D:4 * D] = o2


@jax.jit
def kernel(x, edge_index, edge_feat, W1, b1, W2, b2, W3, b3, W4, b4):
    ef_t = jnp.transpose(edge_feat[:, :2]).reshape(2, 1, E)
    row = edge_index[0]
    col = edge_index[1]
    h, deg = _sc_kernel(x, row, col, ef_t)
    deg_t = jnp.transpose(deg[:, 0, :N])

    w_cat = jnp.concatenate([W1, W2], axis=1)
    b_cat = jnp.concatenate([b1, b2])[None, :]

    out = pl.pallas_call(
        _tc_body,
        grid=(N // BLK,),
        in_specs=[
            pl.BlockSpec((2, BLK, D), lambda i: (0, i, 0)),
            pl.BlockSpec((BLK, 2), lambda i: (i, 0)),
            pl.BlockSpec((BLK, D), lambda i: (i, 0)),
            pl.BlockSpec((D, 2 * D), lambda i: (0, 0)),
            pl.BlockSpec((1, 2 * D), lambda i: (0, 0)),
        ],
        out_specs=pl.BlockSpec((BLK, 4 * D), lambda i: (i, 0)),
        out_shape=jax.ShapeDtypeStruct((N, 4 * D), jnp.float32),
    )(h, deg_t, x, w_cat, b_cat)
    return out


# speedup vs baseline: 39.4320x; 2.4796x over previous
import functools

import jax
import jax.numpy as jnp
from jax import lax
from jax.experimental import pallas as pl
from jax.experimental.pallas import tpu as pltpu
from jax.experimental.pallas import tpu_sc as plsc

N = 10000
E = 320000
D = 128
NSUB = 16
C = 128
NCH0 = (E // C) // NSUB
XCH = (E // C) - NCH0 * NSUB
NP = 10240
RPT = 624
NZ = RPT // 16


def _rsqrt_newton(d):
    i = lax.bitcast_convert_type(d, jnp.int32)
    y = lax.bitcast_convert_type(jnp.int32(0x5F3759DF) - (i >> 1), jnp.float32)
    for _ in range(3):
        y = y * (1.5 - 0.5 * d * y * y)
    return y


def _make_sc_kernel():
    mesh = plsc.VectorSubcoreMesh(core_axis_name="c", subcore_axis_name="s")

    @functools.partial(
        pl.kernel,
        out_type=[
            jax.ShapeDtypeStruct((2, N, D), jnp.float32),
            jax.ShapeDtypeStruct((2, 1, NP), jnp.float32),
        ],
        mesh=mesh,
        scratch_types=[
            pltpu.VMEM((C,), jnp.int32),
            pltpu.VMEM((C,), jnp.int32),
            pltpu.VMEM((C,), jnp.int32),
            pltpu.VMEM((C,), jnp.int32),
            pltpu.VMEM((C,), jnp.int32),
            pltpu.VMEM((C,), jnp.int32),
            pltpu.VMEM((C,), jnp.int32),
            pltpu.VMEM((C,), jnp.int32),
            pltpu.VMEM((C,), jnp.float32),
            pltpu.VMEM((C,), jnp.float32),
            pltpu.VMEM((C,), jnp.float32),
            pltpu.VMEM((C,), jnp.float32),
            pltpu.VMEM((C,), jnp.float32),
            pltpu.VMEM((C,), jnp.float32),
            pltpu.VMEM((C, D), jnp.float32),
            pltpu.VMEM((C, D), jnp.float32),
            pltpu.VMEM((NP,), jnp.float32),
            pltpu.VMEM((16, D), jnp.float32),
            pltpu.VMEM((128,), jnp.float32),
            pltpu.VMEM_SHARED((N, D), jnp.float32),
            pltpu.VMEM_SHARED((NP,), jnp.float32),
            pltpu.SemaphoreType.DMA,
            pltpu.SemaphoreType.DMA,
            pltpu.SemaphoreType.DMA,
            pltpu.SemaphoreType.DMA,
            pltpu.SemaphoreType.DMA,
            pltpu.SemaphoreType.DMA,
            pltpu.SemaphoreType.DMA,
            pltpu.SemaphoreType.DMA,
            pltpu.SemaphoreType.DMA,
        ],
        compiler_params=pltpu.CompilerParams(needs_layout_passes=False),
    )
    def sc_kernel(x_hbm, row_hbm, col_hbm, ef_hbm, h_out, deg_out,
                  c0, c1, c2, v0, v1, v2, sb0, sb1, e0, e1, e2, w0, w1, w2,
                  r0, r1, dis_v, z16r, z128d, h_sh, deg_sh,
                  si0, si1, si2, sg0, sg1, ss0, ss1, ss2, zsem):
        cidx = lax.axis_index("c")
        s = lax.axis_index("s")
        cbuf, vbuf, ebuf, wbuf = [c0, c1, c2], [v0, v1, v2], [e0, e1, e2], [w0, w1, w2]
        sbuf, rbuf = [sb0, sb1], [r0, r1]
        isem, gsem, ssem = [si0, si1, si2], [sg0, sg1], [ss0, ss1, ss2]
        zero16 = jnp.zeros((16,), jnp.float32)

        cb = s * NCH0 + jnp.minimum(s, XCH)
        rbase = s * RPT
        nz = NZ + (s == NSUB - 1).astype(jnp.int32)
        has_extra = s < XCH

        for i in range(16):
            for k in range(8):
                z16r[i, pl.ds(k * 16, 16)] = zero16
        for k in range(8):
            z128d[pl.ds(k * 16, 16)] = zero16

        def zero_issue(t, carry):
            pltpu.async_copy(z16r, h_sh.at[pl.ds(rbase + t * 16, 16)], zsem)
            return carry

        def zero_wait(t, carry):
            pltpu.make_async_copy(z16r, h_sh.at[pl.ds(rbase + t * 16, 16)], zsem).wait()
            return carry

        lax.fori_loop(0, nz, zero_issue, 0)
        for t in range(5):
            pltpu.async_copy(z128d, deg_sh.at[pl.ds((s * 5 + t) * 128, 128)], zsem)
        lax.fori_loop(0, nz, zero_wait, 0)
        for t in range(5):
            pltpu.make_async_copy(z128d, deg_sh.at[pl.ds((s * 5 + t) * 128, 128)], zsem).wait()
        plsc.subcore_barrier()

        def fetch_ce(t, b):
            base = (cb + t) * C
            pltpu.async_copy(col_hbm.at[pl.ds(base, C)], cbuf[b], isem[b])
            pltpu.async_copy(ef_hbm.at[cidx, 0, pl.ds(base, C)], ebuf[b], isem[b])

        def wait_ce(t, b):
            base = (cb + t) * C
            pltpu.make_async_copy(col_hbm.at[pl.ds(base, C)], cbuf[b], isem[b]).wait()
            pltpu.make_async_copy(ef_hbm.at[cidx, 0, pl.ds(base, C)], ebuf[b], isem[b]).wait()

        def fetch_cev(t, b):
            fetch_ce(t, b)
            pltpu.async_copy(row_hbm.at[pl.ds((cb + t) * C, C)], vbuf[b], isem[b])

        def wait_cev(t, b):
            wait_ce(t, b)
            pltpu.make_async_copy(row_hbm.at[pl.ds((cb + t) * C, C)], vbuf[b], isem[b]).wait()

        def sync_cev(t, b):
            base = (cb + t) * C
            pltpu.sync_copy(col_hbm.at[pl.ds(base, C)], cbuf[b])
            pltpu.sync_copy(ef_hbm.at[cidx, 0, pl.ds(base, C)], ebuf[b])
            pltpu.sync_copy(row_hbm.at[pl.ds(base, C)], vbuf[b])

        def absw(b):
            for g in range(8):
                sl = pl.ds(g * 16, 16)
                wbuf[b][sl] = jnp.abs(ebuf[b][sl])

        fetch_ce(0, 0)

        def deg_round(u, carry):
            for b in range(3):
                t = u * 3 + b
                b1 = (b + 1) % 3

                @pl.when(t >= 2)
                def _():
                    pltpu.make_async_copy(wbuf[b1], deg_sh.at[cbuf[b1]], ssem[b1]).wait()

                @pl.when(t + 1 < NCH0)
                def _():
                    fetch_ce(t + 1, b1)

                wait_ce(t, b)
                absw(b)
                pltpu.async_copy(wbuf[b], deg_sh.at[cbuf[b]], ssem[b], add=True)
            return carry

        lax.fori_loop(0, NCH0 // 3, deg_round, 0)
        pltpu.make_async_copy(wbuf[1], deg_sh.at[cbuf[1]], ssem[1]).wait()
        pltpu.make_async_copy(wbuf[2], deg_sh.at[cbuf[2]], ssem[2]).wait()

        @pl.when(has_extra)
        def _():
            base = (cb + NCH0) * C
            pltpu.sync_copy(col_hbm.at[pl.ds(base, C)], cbuf[0])
            pltpu.sync_copy(ef_hbm.at[cidx, 0, pl.ds(base, C)], ebuf[0])
            absw(0)
            pltpu.sync_copy(wbuf[0], deg_sh.at[cbuf[0]], add=True)

        plsc.subcore_barrier()

        pltpu.sync_copy(deg_sh, dis_v)

        @pl.when(s == 0)
        def _():
            pltpu.sync_copy(deg_sh, deg_out.at[cidx, 0])

        def dis_step(i, carry):
            sl = pl.ds(i * 16, 16)
            dis_v[sl] = _rsqrt_newton(dis_v[sl] + 1.0)
            return carry

        lax.fori_loop(0, NP // 16, dis_step, 0)

        def gather(t, bi, br):
            pltpu.async_copy(x_hbm.at[vbuf[bi]], rbuf[br], gsem[br])

        def wait_gather(bi, br):
            pltpu.make_async_copy(x_hbm.at[vbuf[bi]], rbuf[br], gsem[br]).wait()

        def scale(bi, br):
            def group(g, carry):
                r16 = vbuf[bi][pl.ds(g * 16, 16)]
                c16 = cbuf[bi][pl.ds(g * 16, 16)]
                e16 = jnp.abs(ebuf[bi][pl.ds(g * 16, 16)])
                dr = plsc.load_gather(dis_v, [r16])
                dc = plsc.load_gather(dis_v, [c16])
                n16 = dr * e16 * dc
                for l in range(16):
                    spl = jnp.full((16,), n16[l], jnp.float32)
                    j = g * 16 + l
                    for k in range(8):
                        fs = pl.ds(k * 16, 16)
                        rbuf[br][j, fs] = rbuf[br][j, fs] * spl
                return carry

            lax.fori_loop(0, 8, group, 0)

        def copy_col_to_sbuf(bi, br):
            for g in range(8):
                sl = pl.ds(g * 16, 16)
                sbuf[br][sl] = cbuf[bi][sl]

        fetch_cev(0, 0)
        fetch_cev(1, 1)
        wait_cev(0, 0)
        gather(0, 0, 0)

        def edge_round(u, carry):
            for b in range(6):
                t = u * 6 + b
                bi = b % 3
                bi1 = (b + 1) % 3
                bi2 = (b + 2) % 3
                br = b % 2
                br1 = (b + 1) % 2

                @pl.when(t >= 1)
                def _():
                    pltpu.make_async_copy(rbuf[br1], h_sh.at[sbuf[br1]], ssem[br1]).wait()

                @pl.when(t + 1 < NCH0)
                def _():
                    wait_cev(t + 1, bi1)
                    gather(t + 1, bi1, br1)

                @pl.when(t + 2 < NCH0)
                def _():
                    fetch_cev(t + 2, bi2)

                wait_gather(bi, br)
                scale(bi, br)
                copy_col_to_sbuf(bi, br)
                pltpu.async_copy(rbuf[br], h_sh.at[sbuf[br]], ssem[br], add=True)
            return carry

        lax.fori_loop(0, NCH0 // 6, edge_round, 0)
        pltpu.make_async_copy(rbuf[1], h_sh.at[sbuf[1]], ssem[1]).wait()

        @pl.when(has_extra)
        def _():
            sync_cev(NCH0, 0)
            gather(NCH0, 0, 0)
            wait_gather(0, 0)
            scale(0, 0)
            copy_col_to_sbuf(0, 0)
            pltpu.sync_copy(rbuf[0], h_sh.at[sbuf[0]], add=True)

        plsc.subcore_barrier()

        def wb_issue(t, carry):
            off = rbase + t * 16
            pltpu.async_copy(h_sh.at[pl.ds(off, 16)], h_out.at[cidx, pl.ds(off, 16)], zsem)
            return carry

        def wb_wait(t, carry):
            off = rbase + t * 16
            pltpu.make_async_copy(h_sh.at[pl.ds(off, 16)], h_out.at[cidx, pl.ds(off, 16)], zsem).wait()
            return carry

        lax.fori_loop(0, nz, wb_issue, 0)
        lax.fori_loop(0, nz, wb_wait, 0)

    return sc_kernel


_sc_kernel = _make_sc_kernel()

BLK = 1000


def _tc_body(h_ref, deg_ref, x_ref, w_ref, b_ref, o_ref):
    rinv = 1.0 / (deg_ref[...] + 1.0)
    x = x_ref[...]
    t1 = h_ref[0] + x * rinv[:, 0:1]
    t2 = h_ref[1] + x * rinv[:, 1:2]
    o1 = jnp.dot(t1, w_ref[:, :D], preferred_element_type=jnp.float32) + b_ref[0, :D]
    o2 = jnp.dot(t2, w_ref[:, D:], preferred_element_type=jnp.float32) + b_ref[0, D:]
    o_ref[:, 0:D] = o1
    o_ref[:, D:2 * D] = o2
    o_ref[:, 2 * D:3 * D] = o1
    o_ref[:, 3 * D:4 * D] = o2


@jax.jit
def kernel(x, edge_index, edge_feat, W1, b1, W2, b2, W3, b3, W4, b4):
    ef_t = jnp.transpose(edge_feat[:, :2]).reshape(2, 1, E)
    row = edge_index[0]
    col = edge_index[1]
    h, deg = _sc_kernel(x, row, col, ef_t)
    deg_t = jnp.transpose(deg[:, 0, :N])

    w_cat = jnp.concatenate([W1, W2], axis=1)
    b_cat = jnp.concatenate([b1, b2])[None, :]

    out = pl.pallas_call(
        _tc_body,
        grid=(N // BLK,),
        in_specs=[
            pl.BlockSpec((2, BLK, D), lambda i: (0, i, 0)),
            pl.BlockSpec((BLK, 2), lambda i: (i, 0)),
            pl.BlockSpec((BLK, D), lambda i: (i, 0)),
            pl.BlockSpec((D, 2 * D), lambda i: (0, 0)),
            pl.BlockSpec((1, 2 * D), lambda i: (0, 0)),
        ],
        out_specs=pl.BlockSpec((BLK, 4 * D), lambda i: (i, 0)),
        out_shape=jax.ShapeDtypeStruct((N, 4 * D), jnp.float32),
    )(h, deg_t, x, w_cat, b_cat)
    return out


# speedup vs baseline: 39.5608x; 1.0033x over previous
import functools

import jax
import jax.numpy as jnp
from jax import lax
from jax.experimental import pallas as pl
from jax.experimental.pallas import tpu as pltpu
from jax.experimental.pallas import tpu_sc as plsc

N = 10000
E = 320000
D = 128
NSUB = 16
C = 128
NCH0 = (E // C) // NSUB
XCH = (E // C) - NCH0 * NSUB
NP = 10240
RPT = 624
NZ = RPT // 16


def _rsqrt_newton(d):
    i = lax.bitcast_convert_type(d, jnp.int32)
    y = lax.bitcast_convert_type(jnp.int32(0x5F3759DF) - (i >> 1), jnp.float32)
    for _ in range(3):
        y = y * (1.5 - 0.5 * d * y * y)
    return y


def _make_sc_kernel():
    mesh = plsc.VectorSubcoreMesh(core_axis_name="c", subcore_axis_name="s")

    @functools.partial(
        pl.kernel,
        out_type=[
            jax.ShapeDtypeStruct((2, N, D), jnp.float32),
            jax.ShapeDtypeStruct((2, 1, NP), jnp.float32),
        ],
        mesh=mesh,
        scratch_types=[
            pltpu.VMEM((C,), jnp.int32),
            pltpu.VMEM((C,), jnp.int32),
            pltpu.VMEM((C,), jnp.int32),
            pltpu.VMEM((C,), jnp.int32),
            pltpu.VMEM((C,), jnp.int32),
            pltpu.VMEM((C,), jnp.int32),
            pltpu.VMEM((C,), jnp.int32),
            pltpu.VMEM((C,), jnp.int32),
            pltpu.VMEM((C,), jnp.float32),
            pltpu.VMEM((C,), jnp.float32),
            pltpu.VMEM((C,), jnp.float32),
            pltpu.VMEM((C,), jnp.float32),
            pltpu.VMEM((C,), jnp.float32),
            pltpu.VMEM((C,), jnp.float32),
            pltpu.VMEM((C, D), jnp.float32),
            pltpu.VMEM((C, D), jnp.float32),
            pltpu.VMEM((NP,), jnp.float32),
            pltpu.VMEM((16, D), jnp.float32),
            pltpu.VMEM((128,), jnp.float32),
            pltpu.VMEM_SHARED((N, D), jnp.float32),
            pltpu.VMEM_SHARED((NP,), jnp.float32),
            pltpu.SemaphoreType.DMA,
            pltpu.SemaphoreType.DMA,
            pltpu.SemaphoreType.DMA,
            pltpu.SemaphoreType.DMA,
            pltpu.SemaphoreType.DMA,
            pltpu.SemaphoreType.DMA,
            pltpu.SemaphoreType.DMA,
            pltpu.SemaphoreType.DMA,
            pltpu.SemaphoreType.DMA,
        ],
        compiler_params=pltpu.CompilerParams(needs_layout_passes=False),
    )
    def sc_kernel(x_hbm, row_hbm, col_hbm, ef_hbm, h_out, deg_out,
                  c0, c1, c2, v0, v1, v2, sb0, sb1, e0, e1, e2, w0, w1, w2,
                  r0, r1, dis_v, z16r, z128d, h_sh, deg_sh,
                  si0, si1, si2, sg0, sg1, ss0, ss1, ss2, zsem):
        cidx = lax.axis_index("c")
        s = lax.axis_index("s")
        cbuf, vbuf, ebuf, wbuf = [c0, c1, c2], [v0, v1, v2], [e0, e1, e2], [w0, w1, w2]
        sbuf, rbuf = [sb0, sb1], [r0, r1]
        isem, gsem, ssem = [si0, si1, si2], [sg0, sg1], [ss0, ss1, ss2]
        zero16 = jnp.zeros((16,), jnp.float32)

        cb = s * NCH0 + jnp.minimum(s, XCH)
        rbase = s * RPT
        nz = NZ + (s == NSUB - 1).astype(jnp.int32)
        has_extra = s < XCH

        for i in range(16):
            for k in range(8):
                z16r[i, pl.ds(k * 16, 16)] = zero16
        for k in range(8):
            z128d[pl.ds(k * 16, 16)] = zero16

        for t in range(5):
            pltpu.async_copy(z128d, deg_sh.at[pl.ds((s * 5 + t) * 128, 128)], zsem)
        def zero_r0_row(i, carry):
            for k in range(8):
                r0[i, pl.ds(k * 16, 16)] = zero16
            return carry

        lax.fori_loop(0, C, zero_r0_row, 0)
        for t in range(5):
            pltpu.make_async_copy(z128d, deg_sh.at[pl.ds((s * 5 + t) * 128, 128)], zsem).wait()
        plsc.subcore_barrier()

        for t in range(4):
            pltpu.async_copy(r0, h_sh.at[pl.ds(rbase + t * C, C)], zsem)
        pltpu.async_copy(r0.at[pl.ds(0, RPT - 4 * C)], h_sh.at[pl.ds(rbase + 4 * C, RPT - 4 * C)], zsem)

        @pl.when(s == NSUB - 1)
        def _():
            pltpu.async_copy(r0.at[pl.ds(0, 16)], h_sh.at[pl.ds(rbase + RPT, 16)], zsem)

        def fetch_ce(t, b):
            base = (cb + t) * C
            pltpu.async_copy(col_hbm.at[pl.ds(base, C)], cbuf[b], isem[b])
            pltpu.async_copy(ef_hbm.at[cidx, 0, pl.ds(base, C)], ebuf[b], isem[b])

        def wait_ce(t, b):
            base = (cb + t) * C
            pltpu.make_async_copy(col_hbm.at[pl.ds(base, C)], cbuf[b], isem[b]).wait()
            pltpu.make_async_copy(ef_hbm.at[cidx, 0, pl.ds(base, C)], ebuf[b], isem[b]).wait()

        def fetch_cev(t, b):
            fetch_ce(t, b)
            pltpu.async_copy(row_hbm.at[pl.ds((cb + t) * C, C)], vbuf[b], isem[b])

        def wait_cev(t, b):
            wait_ce(t, b)
            pltpu.make_async_copy(row_hbm.at[pl.ds((cb + t) * C, C)], vbuf[b], isem[b]).wait()

        def sync_cev(t, b):
            base = (cb + t) * C
            pltpu.sync_copy(col_hbm.at[pl.ds(base, C)], cbuf[b])
            pltpu.sync_copy(ef_hbm.at[cidx, 0, pl.ds(base, C)], ebuf[b])
            pltpu.sync_copy(row_hbm.at[pl.ds(base, C)], vbuf[b])

        def absw(b):
            for g in range(8):
                sl = pl.ds(g * 16, 16)
                wbuf[b][sl] = jnp.abs(ebuf[b][sl])

        fetch_ce(0, 0)

        def deg_round(u, carry):
            for b in range(3):
                t = u * 3 + b
                b1 = (b + 1) % 3

                @pl.when(t >= 2)
                def _():
                    pltpu.make_async_copy(wbuf[b1], deg_sh.at[cbuf[b1]], ssem[b1]).wait()

                @pl.when(t + 1 < NCH0)
                def _():
                    fetch_ce(t + 1, b1)

                wait_ce(t, b)
                absw(b)
                pltpu.async_copy(wbuf[b], deg_sh.at[cbuf[b]], ssem[b], add=True)
            return carry

        lax.fori_loop(0, NCH0 // 3, deg_round, 0)
        pltpu.make_async_copy(wbuf[1], deg_sh.at[cbuf[1]], ssem[1]).wait()
        pltpu.make_async_copy(wbuf[2], deg_sh.at[cbuf[2]], ssem[2]).wait()

        @pl.when(has_extra)
        def _():
            base = (cb + NCH0) * C
            pltpu.sync_copy(col_hbm.at[pl.ds(base, C)], cbuf[0])
            pltpu.sync_copy(ef_hbm.at[cidx, 0, pl.ds(base, C)], ebuf[0])
            absw(0)
            pltpu.sync_copy(wbuf[0], deg_sh.at[cbuf[0]], add=True)

        for t in range(4):
            pltpu.make_async_copy(r0, h_sh.at[pl.ds(rbase + t * C, C)], zsem).wait()
        pltpu.make_async_copy(r0.at[pl.ds(0, RPT - 4 * C)], h_sh.at[pl.ds(rbase + 4 * C, RPT - 4 * C)], zsem).wait()

        @pl.when(s == NSUB - 1)
        def _():
            pltpu.make_async_copy(r0.at[pl.ds(0, 16)], h_sh.at[pl.ds(rbase + RPT, 16)], zsem).wait()

        plsc.subcore_barrier()

        pltpu.sync_copy(deg_sh, dis_v)

        @pl.when(s == 0)
        def _():
            pltpu.sync_copy(deg_sh, deg_out.at[cidx, 0])

        def dis_step(i, carry):
            sl = pl.ds(i * 16, 16)
            dis_v[sl] = _rsqrt_newton(dis_v[sl] + 1.0)
            return carry

        lax.fori_loop(0, NP // 16, dis_step, 0)

        def gather(t, bi, br):
            pltpu.async_copy(x_hbm.at[vbuf[bi]], rbuf[br], gsem[br])

        def wait_gather(bi, br):
            pltpu.make_async_copy(x_hbm.at[vbuf[bi]], rbuf[br], gsem[br]).wait()

        def scale(bi, br):
            def group(g, carry):
                r16 = vbuf[bi][pl.ds(g * 16, 16)]
                c16 = cbuf[bi][pl.ds(g * 16, 16)]
                e16 = jnp.abs(ebuf[bi][pl.ds(g * 16, 16)])
                dr = plsc.load_gather(dis_v, [r16])
                dc = plsc.load_gather(dis_v, [c16])
                n16 = dr * e16 * dc
                for l in range(16):
                    spl = jnp.full((16,), n16[l], jnp.float32)
                    j = g * 16 + l
                    for k in range(8):
                        fs = pl.ds(k * 16, 16)
                        rbuf[br][j, fs] = rbuf[br][j, fs] * spl
                return carry

            lax.fori_loop(0, 8, group, 0)

        def copy_col_to_sbuf(bi, br):
            for g in range(8):
                sl = pl.ds(g * 16, 16)
                sbuf[br][sl] = cbuf[bi][sl]

        fetch_cev(0, 0)
        fetch_cev(1, 1)
        wait_cev(0, 0)
        gather(0, 0, 0)

        def edge_round(u, carry):
            for b in range(6):
                t = u * 6 + b
                bi = b % 3
                bi1 = (b + 1) % 3
                bi2 = (b + 2) % 3
                br = b % 2
                br1 = (b + 1) % 2

                @pl.when(t >= 1)
                def _():
                    pltpu.make_async_copy(rbuf[br1], h_sh.at[sbuf[br1]], ssem[br1]).wait()

                @pl.when(t + 1 < NCH0)
                def _():
                    wait_cev(t + 1, bi1)
                    gather(t + 1, bi1, br1)

                @pl.when(t + 2 < NCH0)
                def _():
                    fetch_cev(t + 2, bi2)

                wait_gather(bi, br)
                scale(bi, br)
                copy_col_to_sbuf(bi, br)
                pltpu.async_copy(rbuf[br], h_sh.at[sbuf[br]], ssem[br], add=True)
            return carry

        lax.fori_loop(0, NCH0 // 6, edge_round, 0)
        pltpu.make_async_copy(rbuf[1], h_sh.at[sbuf[1]], ssem[1]).wait()

        @pl.when(has_extra)
        def _():
            sync_cev(NCH0, 0)
            gather(NCH0, 0, 0)
            wait_gather(0, 0)
            scale(0, 0)
            copy_col_to_sbuf(0, 0)
            pltpu.sync_copy(rbuf[0], h_sh.at[sbuf[0]], add=True)

        plsc.subcore_barrier()

        pltpu.async_copy(h_sh.at[pl.ds(rbase, RPT)], h_out.at[cidx, pl.ds(rbase, RPT)], zsem)

        @pl.when(s == NSUB - 1)
        def _():
            pltpu.async_copy(h_sh.at[pl.ds(rbase + RPT, 16)],
                             h_out.at[cidx, pl.ds(rbase + RPT, 16)], zsem)

        pltpu.make_async_copy(h_sh.at[pl.ds(rbase, RPT)], h_out.at[cidx, pl.ds(rbase, RPT)], zsem).wait()

        @pl.when(s == NSUB - 1)
        def _():
            pltpu.make_async_copy(h_sh.at[pl.ds(rbase + RPT, 16)],
                                  h_out.at[cidx, pl.ds(rbase + RPT, 16)], zsem).wait()

    return sc_kernel


_sc_kernel = _make_sc_kernel()

BLK = 1000


def _tc_body(h_ref, deg_ref, x_ref, w_ref, b_ref, o_ref):
    rinv = 1.0 / (deg_ref[...] + 1.0)
    x = x_ref[...]
    t1 = h_ref[0] + x * rinv[:, 0:1]
    t2 = h_ref[1] + x * rinv[:, 1:2]
    o1 = jnp.dot(t1, w_ref[:, :D], preferred_element_type=jnp.float32) + b_ref[0, :D]
    o2 = jnp.dot(t2, w_ref[:, D:], preferred_element_type=jnp.float32) + b_ref[0, D:]
    o_ref[:, 0:D] = o1
    o_ref[:, D:2 * D] = o2
    o_ref[:, 2 * D:3 * D] = o1
    o_ref[:, 3 * D:4 * D] = o2


@jax.jit
def kernel(x, edge_index, edge_feat, W1, b1, W2, b2, W3, b3, W4, b4):
    ef_t = jnp.transpose(edge_feat[:, :2]).reshape(2, 1, E)
    row = edge_index[0]
    col = edge_index[1]
    h, deg = _sc_kernel(x, row, col, ef_t)
    deg_t = jnp.transpose(deg[:, 0, :N])

    w_cat = jnp.concatenate([W1, W2], axis=1)
    b_cat = jnp.concatenate([b1, b2])[None, :]

    out = pl.pallas_call(
        _tc_body,
        grid=(N // BLK,),
        in_specs=[
            pl.BlockSpec((2, BLK, D), lambda i: (0, i, 0)),
            pl.BlockSpec((BLK, 2), lambda i: (i, 0)),
            pl.BlockSpec((BLK, D), lambda i: (i, 0)),
            pl.BlockSpec((D, 2 * D), lambda i: (0, 0)),
            pl.BlockSpec((1, 2 * D), lambda i: (0, 0)),
        ],
        out_specs=pl.BlockSpec((BLK, 4 * D), lambda i: (i, 0)),
        out_shape=jax.ShapeDtypeStruct((N, 4 * D), jnp.float32),
    )(h, deg_t, x, w_cat, b_cat)
    return out


# speedup vs baseline: 42.8245x; 1.0825x over previous
import functools

import jax
import jax.numpy as jnp
from jax import lax
from jax.experimental import pallas as pl
from jax.experimental.pallas import tpu as pltpu
from jax.experimental.pallas import tpu_sc as plsc

N = 10000
E = 320000
D = 128
NSUB = 16
C = 128
G = 6
GC = G * C
NCH0 = (E // C) // NSUB
NGRP = NCH0 // G
XCH = (E // C) - NCH0 * NSUB
NP = 10240
RPT = 624


def _rsqrt_newton(d):
    i = lax.bitcast_convert_type(d, jnp.int32)
    y = lax.bitcast_convert_type(jnp.int32(0x5F3759DF) - (i >> 1), jnp.float32)
    for _ in range(3):
        y = y * (1.5 - 0.5 * d * y * y)
    return y


def _make_sc_kernel():
    mesh = plsc.VectorSubcoreMesh(core_axis_name="c", subcore_axis_name="s")

    @functools.partial(
        pl.kernel,
        out_type=[
            jax.ShapeDtypeStruct((2, N, D), jnp.float32),
            jax.ShapeDtypeStruct((2, 1, NP), jnp.float32),
        ],
        mesh=mesh,
        scratch_types=[
            pltpu.VMEM((GC,), jnp.int32),
            pltpu.VMEM((GC,), jnp.int32),
            pltpu.VMEM((GC,), jnp.int32),
            pltpu.VMEM((GC,), jnp.int32),
            pltpu.VMEM((GC,), jnp.float32),
            pltpu.VMEM((GC,), jnp.float32),
            pltpu.VMEM((C,), jnp.int32),
            pltpu.VMEM((C,), jnp.int32),
            pltpu.VMEM((C,), jnp.int32),
            pltpu.VMEM((C,), jnp.int32),
            pltpu.VMEM((C,), jnp.int32),
            pltpu.VMEM((C,), jnp.int32),
            pltpu.VMEM((C,), jnp.int32),
            pltpu.VMEM((C,), jnp.float32),
            pltpu.VMEM((C,), jnp.float32),
            pltpu.VMEM((C,), jnp.float32),
            pltpu.VMEM((C, D), jnp.float32),
            pltpu.VMEM((C, D), jnp.float32),
            pltpu.VMEM((NP,), jnp.float32),
            pltpu.VMEM((128,), jnp.float32),
            pltpu.VMEM_SHARED((N, D), jnp.float32),
            pltpu.VMEM_SHARED((NP,), jnp.float32),
            pltpu.SemaphoreType.DMA,
            pltpu.SemaphoreType.DMA,
            pltpu.SemaphoreType.DMA,
            pltpu.SemaphoreType.DMA,
            pltpu.SemaphoreType.DMA,
            pltpu.SemaphoreType.DMA,
            pltpu.SemaphoreType.DMA,
            pltpu.SemaphoreType.DMA,
        ],
        compiler_params=pltpu.CompilerParams(needs_layout_passes=False),
    )
    def sc_kernel(x_hbm, row_hbm, col_hbm, ef_hbm, h_out, deg_out,
                  cB0, cB1, vB0, vB1, eB0, eB1, cs0, cs1, cs2, gi0, gi1,
                  sb0, sb1, w0, w1, w2, r0, r1, dis_v, z128d,
                  h_sh, deg_sh, si0, si1, sg0, sg1, ss0, ss1, ss2, zsem):
        cidx = lax.axis_index("c")
        s = lax.axis_index("s")
        cB, vB, eB = [cB0, cB1], [vB0, vB1], [eB0, eB1]
        csb, gib, sbuf, wbuf = [cs0, cs1, cs2], [gi0, gi1], [sb0, sb1], [w0, w1, w2]
        rbuf = [r0, r1]
        isem, gsem, ssem = [si0, si1], [sg0, sg1], [ss0, ss1, ss2]
        zero16 = jnp.zeros((16,), jnp.float32)

        cb = s * NCH0 + jnp.minimum(s, XCH)
        rbase = s * RPT
        has_extra = s < XCH

        def fetch_grp_ce(q, a):
            base = (cb + q * G) * C
            pltpu.async_copy(col_hbm.at[pl.ds(base, GC)], cB[a], isem[a])
            pltpu.async_copy(ef_hbm.at[cidx, 0, pl.ds(base, GC)], eB[a], isem[a])

        def wait_grp_ce(q, a):
            base = (cb + q * G) * C
            pltpu.make_async_copy(col_hbm.at[pl.ds(base, GC)], cB[a], isem[a]).wait()
            pltpu.make_async_copy(ef_hbm.at[cidx, 0, pl.ds(base, GC)], eB[a], isem[a]).wait()

        def fetch_grp(q, a):
            fetch_grp_ce(q, a)
            pltpu.async_copy(row_hbm.at[pl.ds((cb + q * G) * C, GC)], vB[a], isem[a])

        def wait_grp(q, a):
            wait_grp_ce(q, a)
            pltpu.make_async_copy(row_hbm.at[pl.ds((cb + q * G) * C, GC)], vB[a], isem[a]).wait()

        def stage(dst, src, off):
            for g in range(8):
                dst[pl.ds(g * 16, 16)] = src[pl.ds(off + g * 16, 16)]

        def stage_absw(w, src, off):
            for g in range(8):
                w[pl.ds(g * 16, 16)] = jnp.abs(src[pl.ds(off + g * 16, 16)])

        fetch_grp_ce(0, 0)

        for k in range(8):
            z128d[pl.ds(k * 16, 16)] = zero16
        for t in range(5):
            pltpu.async_copy(z128d, deg_sh.at[pl.ds((s * 5 + t) * 128, 128)], zsem)

        def zero_r0_row(i, carry):
            for k in range(8):
                r0[i, pl.ds(k * 16, 16)] = zero16
            return carry

        lax.fori_loop(0, C, zero_r0_row, 0)
        for t in range(5):
            pltpu.make_async_copy(z128d, deg_sh.at[pl.ds((s * 5 + t) * 128, 128)], zsem).wait()
        plsc.subcore_barrier()

        for t in range(4):
            pltpu.async_copy(r0, h_sh.at[pl.ds(rbase + t * C, C)], zsem)
        pltpu.async_copy(r0.at[pl.ds(0, RPT - 4 * C)], h_sh.at[pl.ds(rbase + 4 * C, RPT - 4 * C)], zsem)

        @pl.when(s == NSUB - 1)
        def _():
            pltpu.async_copy(r0.at[pl.ds(0, 16)], h_sh.at[pl.ds(rbase + RPT, 16)], zsem)

        def deg_round(u, carry):
            for a in range(2):
                q = 2 * u + a
                a1 = 1 - a

                @pl.when(q + 1 < NGRP)
                def _():
                    fetch_grp_ce(q + 1, a1)

                wait_grp_ce(q, a)
                for j in range(G):
                    t = q * G + j
                    w = j % 3

                    @pl.when(t >= 3)
                    def _():
                        pltpu.make_async_copy(wbuf[w], deg_sh.at[csb[w]], ssem[w]).wait()

                    stage(csb[w], cB[a], j * C)
                    stage_absw(wbuf[w], eB[a], j * C)
                    pltpu.async_copy(wbuf[w], deg_sh.at[csb[w]], ssem[w], add=True)
            return carry

        lax.fori_loop(0, NGRP // 2, deg_round, 0)
        for w in range(3):
            pltpu.make_async_copy(wbuf[w], deg_sh.at[csb[w]], ssem[w]).wait()

        @pl.when(has_extra)
        def _():
            base = (cb + NCH0) * C
            pltpu.sync_copy(col_hbm.at[pl.ds(base, C)], cs0)
            pltpu.sync_copy(ef_hbm.at[cidx, 0, pl.ds(base, C)], w0)
            stage_absw(w1, w0, 0)
            pltpu.sync_copy(w1, deg_sh.at[cs0], add=True)

        for t in range(4):
            pltpu.make_async_copy(r0, h_sh.at[pl.ds(rbase + t * C, C)], zsem).wait()
        pltpu.make_async_copy(r0.at[pl.ds(0, RPT - 4 * C)], h_sh.at[pl.ds(rbase + 4 * C, RPT - 4 * C)], zsem).wait()

        @pl.when(s == NSUB - 1)
        def _():
            pltpu.make_async_copy(r0.at[pl.ds(0, 16)], h_sh.at[pl.ds(rbase + RPT, 16)], zsem).wait()

        plsc.subcore_barrier()

        fetch_grp(0, 0)
        pltpu.sync_copy(deg_sh, dis_v)

        @pl.when(s == 0)
        def _():
            pltpu.sync_copy(deg_sh, deg_out.at[cidx, 0])

        def dis_step(i, carry):
            sl = pl.ds(i * 16, 16)
            dis_v[sl] = _rsqrt_newton(dis_v[sl] + 1.0)
            return carry

        lax.fori_loop(0, NP // 16, dis_step, 0)

        def scale(vb, cbb, eb, off, br):
            def group(g, carry):
                r16 = vb[pl.ds(off + g * 16, 16)]
                c16 = cbb[pl.ds(off + g * 16, 16)]
                e16 = jnp.abs(eb[pl.ds(off + g * 16, 16)])
                dr = plsc.load_gather(dis_v, [r16])
                dc = plsc.load_gather(dis_v, [c16])
                n16 = dr * e16 * dc
                for l in range(16):
                    spl = jnp.full((16,), n16[l], jnp.float32)
                    j = g * 16 + l
                    for k in range(8):
                        fs = pl.ds(k * 16, 16)
                        rbuf[br][j, fs] = rbuf[br][j, fs] * spl
                return carry

            lax.fori_loop(0, 8, group, 0)

        wait_grp(0, 0)
        stage(gi0, vB0, 0)
        pltpu.async_copy(x_hbm.at[gi0], r0, sg0)

        def edge_round(u, carry):
            for a in range(2):
                q = 2 * u + a
                a1 = 1 - a

                @pl.when(q + 1 < NGRP)
                def _():
                    fetch_grp(q + 1, a1)

                for j in range(G):
                    t = q * G + j
                    br = j % 2
                    br1 = 1 - br

                    @pl.when(t >= 1)
                    def _():
                        pltpu.make_async_copy(rbuf[br1], h_sh.at[sbuf[br1]], ssem[br1]).wait()

                    if j < G - 1:
                        stage(gib[br1], vB[a], (j + 1) * C)
                        pltpu.async_copy(x_hbm.at[gib[br1]], rbuf[br1], gsem[br1])
                    else:
                        @pl.when(q + 1 < NGRP)
                        def _():
                            wait_grp(q + 1, a1)
                            stage(gib[br1], vB[a1], 0)
                            pltpu.async_copy(x_hbm.at[gib[br1]], rbuf[br1], gsem[br1])

                    pltpu.make_async_copy(x_hbm.at[gib[br]], rbuf[br], gsem[br]).wait()
                    scale(vB[a], cB[a], eB[a], j * C, br)
                    stage(sbuf[br], cB[a], j * C)
                    pltpu.async_copy(rbuf[br], h_sh.at[sbuf[br]], ssem[br], add=True)
            return carry

        lax.fori_loop(0, NGRP // 2, edge_round, 0)
        pltpu.make_async_copy(rbuf[1], h_sh.at[sbuf[1]], ssem[1]).wait()

        @pl.when(has_extra)
        def _():
            base = (cb + NCH0) * C
            pltpu.sync_copy(col_hbm.at[pl.ds(base, C)], cs0)
            pltpu.sync_copy(ef_hbm.at[cidx, 0, pl.ds(base, C)], w0)
            pltpu.sync_copy(row_hbm.at[pl.ds(base, C)], gi0)
            pltpu.async_copy(x_hbm.at[gi0], r0, sg0)
            pltpu.make_async_copy(x_hbm.at[gi0], r0, sg0).wait()
            scale(gi0, cs0, w0, 0, 0)
            pltpu.sync_copy(r0, h_sh.at[cs0], add=True)

        plsc.subcore_barrier()

        pltpu.async_copy(h_sh.at[pl.ds(rbase, RPT)], h_out.at[cidx, pl.ds(rbase, RPT)], zsem)

        @pl.when(s == NSUB - 1)
        def _():
            pltpu.async_copy(h_sh.at[pl.ds(rbase + RPT, 16)],
                             h_out.at[cidx, pl.ds(rbase + RPT, 16)], zsem)

        pltpu.make_async_copy(h_sh.at[pl.ds(rbase, RPT)], h_out.at[cidx, pl.ds(rbase, RPT)], zsem).wait()

        @pl.when(s == NSUB - 1)
        def _():
            pltpu.make_async_copy(h_sh.at[pl.ds(rbase + RPT, 16)],
                                  h_out.at[cidx, pl.ds(rbase + RPT, 16)], zsem).wait()

    return sc_kernel


_sc_kernel = _make_sc_kernel()

BLK = 1000


def _tc_body(h_ref, deg_ref, x_ref, w_ref, b_ref, o_ref):
    rinv = 1.0 / (deg_ref[...] + 1.0)
    x = x_ref[...]
    t1 = h_ref[0] + x * rinv[:, 0:1]
    t2 = h_ref[1] + x * rinv[:, 1:2]
    o1 = jnp.dot(t1, w_ref[:, :D], preferred_element_type=jnp.float32) + b_ref[0, :D]
    o2 = jnp.dot(t2, w_ref[:, D:], preferred_element_type=jnp.float32) + b_ref[0, D:]
    o_ref[:, 0:D] = o1
    o_ref[:, D:2 * D] = o2
    o_ref[:, 2 * D:3 * D] = o1
    o_ref[:, 3 * D:4 * D] = o2


@jax.jit
def kernel(x, edge_index, edge_feat, W1, b1, W2, b2, W3, b3, W4, b4):
    ef_t = jnp.transpose(edge_feat[:, :2]).reshape(2, 1, E)
    row = edge_index[0]
    col = edge_index[1]
    h, deg = _sc_kernel(x, row, col, ef_t)
    deg_t = jnp.transpose(deg[:, 0, :N])

    w_cat = jnp.concatenate([W1, W2], axis=1)
    b_cat = jnp.concatenate([b1, b2])[None, :]

    out = pl.pallas_call(
        _tc_body,
        grid=(N // BLK,),
        in_specs=[
            pl.BlockSpec((2, BLK, D), lambda i: (0, i, 0)),
            pl.BlockSpec((BLK, 2), lambda i: (i, 0)),
            pl.BlockSpec((BLK, D), lambda i: (i, 0)),
            pl.BlockSpec((D, 2 * D), lambda i: (0, 0)),
            pl.BlockSpec((1, 2 * D), lambda i: (0, 0)),
        ],
        out_specs=pl.BlockSpec((BLK, 4 * D), lambda i: (i, 0)),
        out_shape=jax.ShapeDtypeStruct((N, 4 * D), jnp.float32),
    )(h, deg_t, x, w_cat, b_cat)
    return out
